# Initial kernel scaffold; baseline (speedup 1.0000x reference)
import functools

import jax
import jax.numpy as jnp
from jax import lax
from jax.experimental import pallas as pl
from jax.experimental.pallas import tpu as pltpu
from jax.experimental.pallas import tpu_sc as plsc

_B = 16384
_NF = 26
_V = 100000
_D = 32
_H = 256
_O = 1

_NC = 2
_NS = 16
_NW = _NC * _NS

_ROWS = _B * _NF
_ROWS_W = _ROWS // _NW
_CHUNK = 128
_NCHUNK = _ROWS_W // _CHUNK
_GROUP = 8
_GROUP_ROWS = _CHUNK * _GROUP
_NGROUP = _NCHUNK // _GROUP


def _gather_body(table_hbm, idx_hbm, out_hbm, idx_v, rows_v, sem):
    wid = lax.axis_index("s") * _NC + lax.axis_index("c")
    base = wid * _ROWS_W
    pltpu.sync_copy(idx_hbm.at[wid], idx_v)

    def group(g, carry):
        cbase = g * _GROUP
        copies = []
        for j in range(_GROUP):
            cp = pltpu.async_copy(
                table_hbm.at[idx_v.at[cbase + j]],
                rows_v.at[pl.ds(j * _CHUNK, _CHUNK)],
                sem,
            )
            copies.append(cp)
        for cp in copies:
            cp.wait()
        pltpu.sync_copy(
            rows_v, out_hbm.at[pl.ds(base + g * _GROUP_ROWS, _GROUP_ROWS)]
        )
        return carry

    lax.fori_loop(0, _NGROUP, group, 0)


def _sc_gather(table_flat, idx3):
    mesh = plsc.VectorSubcoreMesh(core_axis_name="c", subcore_axis_name="s")
    f = pl.kernel(
        _gather_body,
        mesh=mesh,
        out_type=jax.ShapeDtypeStruct((_ROWS, _D), jnp.float32),
        scratch_types=[
            pltpu.VMEM((_NCHUNK, _CHUNK), jnp.int32),
            pltpu.VMEM((_GROUP_ROWS, _D), jnp.float32),
            pltpu.SemaphoreType.DMA,
        ],
    )
    return f(table_flat, idx3)


_BB = 512


def _mlp_body(emb_ref, w1_ref, b1_ref, w2_ref, b2_ref, out_ref):
    h = jnp.dot(emb_ref[...], w1_ref[...], preferred_element_type=jnp.float32)
    h = jnp.maximum(h + b1_ref[...], 0.0)
    o = jnp.dot(h, w2_ref[...], preferred_element_type=jnp.float32)
    out_ref[...] = jax.nn.sigmoid(o + b2_ref[...])


def _tc_mlp(emb, W1, b1, W2, b2):
    grid = (_B // _BB,)
    return pl.pallas_call(
        _mlp_body,
        grid=grid,
        in_specs=[
            pl.BlockSpec((_BB, _NF * _D), lambda i: (i, 0)),
            pl.BlockSpec((_NF * _D, _H), lambda i: (0, 0)),
            pl.BlockSpec((1, _H), lambda i: (0, 0)),
            pl.BlockSpec((_H, _O), lambda i: (0, 0)),
            pl.BlockSpec((1, _O), lambda i: (0, 0)),
        ],
        out_specs=pl.BlockSpec((_BB, _O), lambda i: (i, 0)),
        out_shape=jax.ShapeDtypeStruct((_B, _O), jnp.float32),
    )(emb, W1, b1.reshape(1, _H), W2, b2.reshape(1, _O))


def kernel(x, tables, W1, b1, W2, b2):
    offs = (jnp.arange(_NF, dtype=jnp.int32) * _V)[None, :]
    flat_idx = (x.astype(jnp.int32) + offs).reshape(_NW, _NCHUNK, _CHUNK)
    table_flat = tables.reshape(_NF * _V, _D)
    emb_flat = _sc_gather(table_flat, flat_idx)
    emb = emb_flat.reshape(_B, _NF * _D)
    return _tc_mlp(emb, W1, b1, W2, b2)


# speedup vs baseline: 8.0018x; 8.0018x over previous
import functools

import jax
import jax.numpy as jnp
from jax import lax
from jax.experimental import pallas as pl
from jax.experimental.pallas import tpu as pltpu
from jax.experimental.pallas import tpu_sc as plsc

_B = 16384
_NF = 26
_V = 100000
_D = 32
_H = 256
_O = 1

_NC = 2
_NS = 16
_NW = _NC * _NS

_ROWS = _B * _NF
_ROWS_W = _ROWS // _NW
_CHUNK = 128
_NCHUNK = _ROWS_W // _CHUNK
_GROUP = 8
_GROUP_ROWS = _CHUNK * _GROUP
_NGROUP = _NCHUNK // _GROUP


def _gather_body(table_hbm, idx_hbm, out_hbm, idx_v, rows_v, sem):
    wid = lax.axis_index("s") * _NC + lax.axis_index("c")
    base = wid * _ROWS_W
    pltpu.sync_copy(idx_hbm.at[wid], idx_v)

    def group(g, carry):
        cbase = g * _GROUP
        copies = []
        for j in range(_GROUP):
            cp = pltpu.async_copy(
                table_hbm.at[idx_v.at[cbase + j]],
                rows_v.at[pl.ds(j * _CHUNK, _CHUNK)],
                sem,
            )
            copies.append(cp)
        for cp in copies:
            cp.wait()
        pltpu.sync_copy(
            rows_v, out_hbm.at[pl.ds(base + g * _GROUP_ROWS, _GROUP_ROWS)]
        )
        return carry

    lax.fori_loop(0, _NGROUP, group, 0)


def _sc_gather(table_flat, idx3):
    mesh = plsc.VectorSubcoreMesh(core_axis_name="c", subcore_axis_name="s")
    f = pl.kernel(
        _gather_body,
        mesh=mesh,
        out_type=jax.ShapeDtypeStruct((_ROWS, _D), jnp.float32),
        scratch_types=[
            pltpu.VMEM((_NCHUNK, _CHUNK), jnp.int32),
            pltpu.VMEM((_GROUP_ROWS, _D), jnp.float32),
            pltpu.SemaphoreType.DMA,
        ],
        compiler_params=pltpu.CompilerParams(use_tc_tiling_on_sc=False),
    )
    return f(table_flat, idx3)


_BB = 512


def _mlp_body(emb_ref, w1_ref, b1_ref, w2_ref, b2_ref, out_ref):
    h = jnp.dot(emb_ref[...], w1_ref[...], preferred_element_type=jnp.float32)
    h = jnp.maximum(h + b1_ref[...], 0.0)
    o = jnp.dot(h, w2_ref[...], preferred_element_type=jnp.float32)
    out_ref[...] = jax.nn.sigmoid(o + b2_ref[...])


def _tc_mlp(emb, W1, b1, W2, b2):
    grid = (_B // _BB,)
    return pl.pallas_call(
        _mlp_body,
        grid=grid,
        in_specs=[
            pl.BlockSpec((_BB, _NF * _D), lambda i: (i, 0)),
            pl.BlockSpec((_NF * _D, _H), lambda i: (0, 0)),
            pl.BlockSpec((1, _H), lambda i: (0, 0)),
            pl.BlockSpec((_H, _O), lambda i: (0, 0)),
            pl.BlockSpec((1, _O), lambda i: (0, 0)),
        ],
        out_specs=pl.BlockSpec((_BB, _O), lambda i: (i, 0)),
        out_shape=jax.ShapeDtypeStruct((_B, _O), jnp.float32),
    )(emb, W1, b1.reshape(1, _H), W2, b2.reshape(1, _O))


def kernel(x, tables, W1, b1, W2, b2):
    offs = (jnp.arange(_NF, dtype=jnp.int32) * _V)[None, :]
    flat_idx = (x.astype(jnp.int32) + offs).reshape(_NW, _NCHUNK, _CHUNK)
    table_flat = tables.reshape(_NF * _V, _D)
    emb_flat = _sc_gather(table_flat, flat_idx)
    emb = emb_flat.reshape(_B, _NF * _D)
    return _tc_mlp(emb, W1, b1, W2, b2)
